# Initial kernel scaffold; baseline (speedup 1.0000x reference)
import jax
import jax.numpy as jnp
from jax.experimental import pallas as pl


def kernel(intermediate_states, down_weight, full_topk_ids, full_topk_weight):
    raise NotImplementedError("write your pallas kernel here")


# speedup vs baseline: 3.7347x; 3.7347x over previous
import functools

import jax
import jax.numpy as jnp
from jax import lax
from jax.experimental import pallas as pl
from jax.experimental.pallas import tpu as pltpu
from jax.experimental.pallas import tpu_sc as plsc

_SC_CORES = 2
_SC_SUBCORES = 16
_SC_WORKERS = _SC_CORES * _SC_SUBCORES

_BLK = 32


def _prescale_body(x_ref, w_ref, o_ref):
    o_ref[:] = x_ref[:] * w_ref[:]


def _prescale(x, w):
    T, K = x.shape
    rb = 256
    return pl.pallas_call(
        _prescale_body,
        grid=(T // rb,),
        in_specs=[
            pl.BlockSpec((rb, K), lambda i: (i, 0)),
            pl.BlockSpec((rb, 1), lambda i: (i, 0)),
        ],
        out_specs=pl.BlockSpec((rb, K), lambda i: (i, 0)),
        out_shape=jax.ShapeDtypeStruct((T, K), x.dtype),
    )(x, w)


def _make_sc_row_gather(T_rows, D, R):
    per_w = T_rows // _SC_WORKERS
    n_chunks = per_w // R
    mesh = plsc.VectorSubcoreMesh(core_axis_name="c", subcore_axis_name="s")

    @functools.partial(
        pl.kernel,
        out_type=jax.ShapeDtypeStruct((T_rows, D), jnp.float32),
        mesh=mesh,
        scratch_types=[
            pltpu.VMEM((R,), jnp.int32),
            pltpu.VMEM((R, D), jnp.float32),
            pltpu.SemaphoreType.DMA,
        ],
    )
    def gather_kernel(src_hbm, idx_hbm, out_hbm, idx_v, rows_v, sem):
        wid = lax.axis_index("s") * _SC_CORES + lax.axis_index("c")
        for c in range(n_chunks):
            base = wid * per_w + c * R
            pltpu.sync_copy(idx_hbm.at[pl.ds(base, R)], idx_v)
            pltpu.async_copy(src_hbm.at[idx_v], rows_v, sem).wait()
            pltpu.sync_copy(rows_v, out_hbm.at[pl.ds(base, R)])

    return gather_kernel


def _ragged_matmul_body(blk_r, exp_r, lo_r, hi_r, xs_ref, w_ref, o_ref):
    s = pl.program_id(0)
    lo = lo_r[s]
    hi = hi_r[s]

    @pl.when(hi > lo)
    def _():
        y = jnp.dot(xs_ref[:], w_ref[0], preferred_element_type=jnp.float32)
        rows = lax.broadcasted_iota(jnp.int32, (xs_ref.shape[0], 1), 0)
        mask = (rows >= lo) & (rows < hi)
        o_ref[:] = jnp.where(mask, y, o_ref[:])


def _ragged_matmul(xs, W, step_blk, step_exp, step_lo, step_hi, n_steps):
    T, K = xs.shape
    E, _, H = W.shape
    grid_spec = pltpu.PrefetchScalarGridSpec(
        num_scalar_prefetch=4,
        grid=(n_steps,),
        in_specs=[
            pl.BlockSpec((_BLK, K), lambda s, blk, exp, lo, hi: (blk[s], 0)),
            pl.BlockSpec((1, K, H), lambda s, blk, exp, lo, hi: (exp[s], 0, 0)),
        ],
        out_specs=pl.BlockSpec((_BLK, H), lambda s, blk, exp, lo, hi: (blk[s], 0)),
    )
    return pl.pallas_call(
        _ragged_matmul_body,
        grid_spec=grid_spec,
        out_shape=jax.ShapeDtypeStruct((T, H), jnp.float32),
    )(step_blk, step_exp, step_lo, step_hi, xs, W)


def kernel(intermediate_states, down_weight, full_topk_ids, full_topk_weight):
    x = intermediate_states
    W = down_weight
    T, K = x.shape
    E, _, H = W.shape

    flat_ids = full_topk_ids.reshape(T).astype(jnp.int32)
    order = jnp.argsort(flat_ids).astype(jnp.int32)
    sorted_ids = flat_ids[order]
    inv_order = jnp.argsort(order).astype(jnp.int32)
    offsets = jnp.searchsorted(
        sorted_ids, jnp.arange(E, dtype=jnp.int32), side="left"
    ).astype(jnp.int32)

    nb = T // _BLK
    n_steps = nb + E
    bounds = jnp.sort(
        jnp.concatenate([jnp.arange(nb, dtype=jnp.int32) * _BLK, offsets])
    )
    seg_start = bounds
    seg_end = jnp.concatenate([bounds[1:], jnp.array([T], jnp.int32)])
    cl = jnp.clip(seg_start, 0, T - 1)
    step_blk = cl // _BLK
    step_exp = sorted_ids[cl]
    step_lo = jnp.clip(seg_start - step_blk * _BLK, 0, _BLK)
    step_hi = jnp.clip(seg_end - step_blk * _BLK, 0, _BLK)

    xw = _prescale(x, full_topk_weight.astype(jnp.float32))
    xs = _make_sc_row_gather(T, K, 64)(xw, order)
    ys = _ragged_matmul(xs, W, step_blk, step_exp, step_lo, step_hi, n_steps)
    out = _make_sc_row_gather(T, H, 32)(ys, inv_order)
    return out


# speedup vs baseline: 5.1357x; 1.3751x over previous
import functools

import jax
import jax.numpy as jnp
from jax import lax
from jax.experimental import pallas as pl
from jax.experimental.pallas import tpu as pltpu
from jax.experimental.pallas import tpu_sc as plsc

_SC_CORES = 2
_SC_SUBCORES = 16
_SC_WORKERS = _SC_CORES * _SC_SUBCORES

_BLK = 32


def _prescale_body(x_ref, w_ref, o_ref):
    o_ref[:] = x_ref[:] * w_ref[:]


def _prescale(x, w):
    T, K = x.shape
    rb = 256
    return pl.pallas_call(
        _prescale_body,
        grid=(T // rb,),
        in_specs=[
            pl.BlockSpec((rb, K), lambda i: (i, 0)),
            pl.BlockSpec((rb, 1), lambda i: (i, 0)),
        ],
        out_specs=pl.BlockSpec((rb, K), lambda i: (i, 0)),
        out_shape=jax.ShapeDtypeStruct((T, K), x.dtype),
    )(x, w)


def _make_sc_row_gather(T_rows, D, R):
    per_w = T_rows // _SC_WORKERS
    n_chunks = per_w // R
    mesh = plsc.VectorSubcoreMesh(core_axis_name="c", subcore_axis_name="s")

    @functools.partial(
        pl.kernel,
        out_type=jax.ShapeDtypeStruct((T_rows, D), jnp.float32),
        mesh=mesh,
        scratch_types=[
            pltpu.VMEM((R,), jnp.int32),
            pltpu.VMEM((R, D), jnp.float32),
            pltpu.SemaphoreType.DMA,
        ],
    )
    def gather_kernel(src_hbm, idx_hbm, out_hbm, idx_v, rows_v, sem):
        wid = lax.axis_index("s") * _SC_CORES + lax.axis_index("c")
        for c in range(n_chunks):
            base = wid * per_w + c * R
            pltpu.sync_copy(idx_hbm.at[pl.ds(base, R)], idx_v)
            pltpu.async_copy(src_hbm.at[idx_v], rows_v, sem).wait()
            pltpu.sync_copy(rows_v, out_hbm.at[pl.ds(base, R)])

    return gather_kernel


def _ragged_matmul_body(off_r, xs_ref, w_ref, o_ref):
    e = pl.program_id(0)
    T = xs_ref.shape[0]
    start = off_r[e]
    end = off_r[e + 1]
    blk0 = start // _BLK
    n_chunks = (end + _BLK - 1) // _BLK - blk0

    def chunk(j, carry):
        s0 = (blk0 + j) * _BLK
        y = jnp.dot(
            xs_ref[pl.ds(s0, _BLK), :], w_ref[0],
            preferred_element_type=jnp.float32,
        )
        r = s0 + lax.broadcasted_iota(jnp.int32, (_BLK, 1), 0)
        mask = (r >= start) & (r < end)
        o_ref[pl.ds(s0, _BLK), :] = jnp.where(mask, y, o_ref[pl.ds(s0, _BLK), :])
        return carry

    lax.fori_loop(0, n_chunks, chunk, 0)


def _ragged_matmul(xs, W, offsets_ext):
    T, K = xs.shape
    E, _, H = W.shape
    grid_spec = pltpu.PrefetchScalarGridSpec(
        num_scalar_prefetch=1,
        grid=(E,),
        in_specs=[
            pl.BlockSpec((T, K), lambda e, off: (0, 0)),
            pl.BlockSpec((1, K, H), lambda e, off: (e, 0, 0)),
        ],
        out_specs=pl.BlockSpec((T, H), lambda e, off: (0, 0)),
    )
    return pl.pallas_call(
        _ragged_matmul_body,
        grid_spec=grid_spec,
        out_shape=jax.ShapeDtypeStruct((T, H), jnp.float32),
    )(offsets_ext, xs, W)


def kernel(intermediate_states, down_weight, full_topk_ids, full_topk_weight):
    x = intermediate_states
    W = down_weight
    T, K = x.shape
    E, _, H = W.shape

    flat_ids = full_topk_ids.reshape(T).astype(jnp.int32)
    order = jnp.argsort(flat_ids).astype(jnp.int32)
    sorted_ids = flat_ids[order]
    inv_order = jnp.argsort(order).astype(jnp.int32)
    offsets_ext = jnp.searchsorted(
        sorted_ids, jnp.arange(E + 1, dtype=jnp.int32), side="left"
    ).astype(jnp.int32)

    xw = _prescale(x, full_topk_weight.astype(jnp.float32))
    xs = _make_sc_row_gather(T, K, 64)(xw, order)
    ys = _ragged_matmul(xs, W, offsets_ext)
    out = _make_sc_row_gather(T, H, 32)(ys, inv_order)
    return out


# speedup vs baseline: 5.3651x; 1.0447x over previous
import functools

import jax
import jax.numpy as jnp
from jax import lax
from jax.experimental import pallas as pl
from jax.experimental.pallas import tpu as pltpu
from jax.experimental.pallas import tpu_sc as plsc

_SC_CORES = 2
_SC_SUBCORES = 16
_SC_WORKERS = _SC_CORES * _SC_SUBCORES

_BLK = 32


def _make_sc_row_gather(T_rows, D, R):
    per_w = T_rows // _SC_WORKERS
    n_chunks = per_w // R
    mesh = plsc.VectorSubcoreMesh(core_axis_name="c", subcore_axis_name="s")

    @functools.partial(
        pl.kernel,
        out_type=jax.ShapeDtypeStruct((T_rows, D), jnp.float32),
        mesh=mesh,
        scratch_types=[
            pltpu.VMEM((R,), jnp.int32),
            pltpu.VMEM((R, D), jnp.float32),
            pltpu.SemaphoreType.DMA,
        ],
    )
    def gather_kernel(src_hbm, idx_hbm, out_hbm, idx_v, rows_v, sem):
        wid = lax.axis_index("s") * _SC_CORES + lax.axis_index("c")
        for c in range(n_chunks):
            base = wid * per_w + c * R
            pltpu.sync_copy(idx_hbm.at[pl.ds(base, R)], idx_v)
            pltpu.async_copy(src_hbm.at[idx_v], rows_v, sem).wait()
            pltpu.sync_copy(rows_v, out_hbm.at[pl.ds(base, R)])

    return gather_kernel


def _make_sc_row_and_scalar_gather(T_rows, D, R):
    per_w = T_rows // _SC_WORKERS
    n_chunks = per_w // R
    lanes = 128
    mesh = plsc.VectorSubcoreMesh(core_axis_name="c", subcore_axis_name="s")

    @functools.partial(
        pl.kernel,
        out_type=(
            jax.ShapeDtypeStruct((T_rows, D), jnp.float32),
            jax.ShapeDtypeStruct((T_rows, lanes), jnp.float32),
        ),
        mesh=mesh,
        scratch_types=[
            pltpu.VMEM((R,), jnp.int32),
            pltpu.VMEM((R, D), jnp.float32),
            pltpu.VMEM((R, lanes), jnp.float32),
            pltpu.SemaphoreType.DMA,
            pltpu.SemaphoreType.DMA,
        ],
    )
    def gather_kernel(src_hbm, idx_hbm, svec_hbm, rows_out, s_out,
                      idx_v, rows_v, sg_v, sem, sem2):
        wid = lax.axis_index("s") * _SC_CORES + lax.axis_index("c")
        for c in range(n_chunks):
            base = wid * per_w + c * R
            pltpu.sync_copy(idx_hbm.at[pl.ds(base, R)], idx_v)
            cp = pltpu.async_copy(src_hbm.at[idx_v], rows_v, sem)
            cp2 = pltpu.async_copy(svec_hbm.at[idx_v], sg_v, sem2)
            cp.wait()
            cp2.wait()
            pltpu.sync_copy(rows_v, rows_out.at[pl.ds(base, R)])
            pltpu.sync_copy(sg_v, s_out.at[pl.ds(base, R)])

    return gather_kernel


def _ragged_matmul_body(off_r, xs_ref, w_ref, ws_ref, o_ref):
    e = pl.program_id(0)
    start = off_r[e]
    end = off_r[e + 1]
    blk0 = start // _BLK
    n_chunks = (end + _BLK - 1) // _BLK - blk0

    def chunk(j, carry):
        s0 = (blk0 + j) * _BLK
        y = jnp.dot(
            xs_ref[pl.ds(s0, _BLK), :], w_ref[0],
            preferred_element_type=jnp.float32,
        )
        y = y * ws_ref[pl.ds(s0, _BLK), :]
        r = s0 + lax.broadcasted_iota(jnp.int32, (_BLK, 1), 0)
        mask = (r >= start) & (r < end)
        o_ref[pl.ds(s0, _BLK), :] = jnp.where(mask, y, o_ref[pl.ds(s0, _BLK), :])
        return carry

    lax.fori_loop(0, n_chunks, chunk, 0)


def _ragged_matmul(xs, W, ws, offsets_ext):
    T, K = xs.shape
    E, _, H = W.shape
    grid_spec = pltpu.PrefetchScalarGridSpec(
        num_scalar_prefetch=1,
        grid=(E,),
        in_specs=[
            pl.BlockSpec((T, K), lambda e, off: (0, 0)),
            pl.BlockSpec((1, K, H), lambda e, off: (e, 0, 0)),
            pl.BlockSpec((T, 1), lambda e, off: (0, 0)),
        ],
        out_specs=pl.BlockSpec((T, H), lambda e, off: (0, 0)),
    )
    return pl.pallas_call(
        _ragged_matmul_body,
        grid_spec=grid_spec,
        out_shape=jax.ShapeDtypeStruct((T, H), jnp.float32),
    )(offsets_ext, xs, W, ws)


def kernel(intermediate_states, down_weight, full_topk_ids, full_topk_weight):
    x = intermediate_states
    W = down_weight
    T, K = x.shape
    E, _, H = W.shape

    flat_ids = full_topk_ids.reshape(T).astype(jnp.int32)
    order = jnp.argsort(flat_ids).astype(jnp.int32)
    sorted_ids = flat_ids[order]
    inv_order = jnp.argsort(order).astype(jnp.int32)
    offsets_ext = jnp.searchsorted(
        sorted_ids, jnp.arange(E + 1, dtype=jnp.int32), side="left"
    ).astype(jnp.int32)

    w16 = jnp.broadcast_to(
        full_topk_weight.astype(jnp.float32).reshape(T, 1), (T, 128)
    )
    xs, ws16 = _make_sc_row_and_scalar_gather(T, K, 64)(x, order, w16)
    ys = _ragged_matmul(xs, W, ws16[:, :1], offsets_ext)
    out = _make_sc_row_gather(T, H, 32)(ys, inv_order)
    return out


# speedup vs baseline: 5.6956x; 1.0616x over previous
import functools

import jax
import jax.numpy as jnp
from jax import lax
from jax.experimental import pallas as pl
from jax.experimental.pallas import tpu as pltpu
from jax.experimental.pallas import tpu_sc as plsc

_SC_CORES = 2
_SC_SUBCORES = 16
_SC_WORKERS = _SC_CORES * _SC_SUBCORES

_BLK = 32


def _make_sc_row_gather(T_rows, D, R):
    per_w = T_rows // _SC_WORKERS
    n_chunks = per_w // R
    mesh = plsc.VectorSubcoreMesh(core_axis_name="c", subcore_axis_name="s")

    @functools.partial(
        pl.kernel,
        out_type=jax.ShapeDtypeStruct((T_rows, D), jnp.float32),
        mesh=mesh,
        scratch_types=[
            pltpu.VMEM((R,), jnp.int32),
            pltpu.VMEM((R, D), jnp.float32),
            pltpu.SemaphoreType.DMA,
        ],
    )
    def gather_kernel(src_hbm, idx_hbm, out_hbm, idx_v, rows_v, sem):
        wid = lax.axis_index("s") * _SC_CORES + lax.axis_index("c")
        for c in range(n_chunks):
            base = wid * per_w + c * R
            pltpu.sync_copy(idx_hbm.at[pl.ds(base, R)], idx_v)
            pltpu.async_copy(src_hbm.at[idx_v], rows_v, sem).wait()
            pltpu.sync_copy(rows_v, out_hbm.at[pl.ds(base, R)])

    return gather_kernel


def _make_sc_row_and_scalar_gather(T_rows, D, R):
    per_w = T_rows // _SC_WORKERS
    n_chunks = per_w // R
    lanes = 128
    mesh = plsc.VectorSubcoreMesh(core_axis_name="c", subcore_axis_name="s")

    @functools.partial(
        pl.kernel,
        out_type=(
            jax.ShapeDtypeStruct((T_rows, D), jnp.float32),
            jax.ShapeDtypeStruct((T_rows, lanes), jnp.float32),
        ),
        mesh=mesh,
        scratch_types=[
            pltpu.VMEM((R,), jnp.int32),
            pltpu.VMEM((R, D), jnp.float32),
            pltpu.VMEM((R, lanes), jnp.float32),
            pltpu.SemaphoreType.DMA,
            pltpu.SemaphoreType.DMA,
        ],
    )
    def gather_kernel(src_hbm, idx_hbm, svec_hbm, rows_out, s_out,
                      idx_v, rows_v, sg_v, sem, sem2):
        wid = lax.axis_index("s") * _SC_CORES + lax.axis_index("c")
        for c in range(n_chunks):
            base = wid * per_w + c * R
            pltpu.sync_copy(idx_hbm.at[pl.ds(base, R)], idx_v)
            cp = pltpu.async_copy(src_hbm.at[idx_v], rows_v, sem)
            cp2 = pltpu.async_copy(svec_hbm.at[idx_v], sg_v, sem2)
            cp.wait()
            cp2.wait()
            pltpu.sync_copy(rows_v, rows_out.at[pl.ds(base, R)])
            pltpu.sync_copy(sg_v, s_out.at[pl.ds(base, R)])

    return gather_kernel


def _ragged_matmul_body(off_r, xs_ref, w_ref, ws_ref, o_ref):
    e = pl.program_id(0)
    start = off_r[e]
    end = off_r[e + 1]
    blk0 = start // _BLK
    n_chunks = (end + _BLK - 1) // _BLK - blk0

    def chunk(j, carry):
        s0 = (blk0 + j) * _BLK
        y = jnp.dot(
            xs_ref[pl.ds(s0, _BLK), :], w_ref[0],
            preferred_element_type=jnp.float32,
        )
        y = y * ws_ref[pl.ds(s0, _BLK), :]
        r = s0 + lax.broadcasted_iota(jnp.int32, (_BLK, 1), 0)
        mask = (r >= start) & (r < end)
        o_ref[pl.ds(s0, _BLK), :] = jnp.where(mask, y, o_ref[pl.ds(s0, _BLK), :])
        return carry

    lax.fori_loop(0, n_chunks, chunk, 0)


def _ragged_matmul(xs, W, ws, offsets_ext):
    T, K = xs.shape
    E, _, H = W.shape
    grid_spec = pltpu.PrefetchScalarGridSpec(
        num_scalar_prefetch=1,
        grid=(E,),
        in_specs=[
            pl.BlockSpec((T, K), lambda e, off: (0, 0)),
            pl.BlockSpec((1, K, H), lambda e, off: (e, 0, 0)),
            pl.BlockSpec((T, 1), lambda e, off: (0, 0)),
        ],
        out_specs=pl.BlockSpec((T, H), lambda e, off: (0, 0)),
    )
    return pl.pallas_call(
        _ragged_matmul_body,
        grid_spec=grid_spec,
        out_shape=jax.ShapeDtypeStruct((T, H), jnp.float32),
    )(offsets_ext, xs, W, ws)


def kernel(intermediate_states, down_weight, full_topk_ids, full_topk_weight):
    x = intermediate_states
    W = down_weight
    T, K = x.shape
    E, _, H = W.shape

    flat_ids = full_topk_ids.reshape(T).astype(jnp.int32)
    order = jnp.argsort(flat_ids).astype(jnp.int32)
    cmp = flat_ids[None, :] < jnp.arange(1, E + 1, dtype=jnp.int32)[:, None]
    offsets_ext = jnp.concatenate(
        [jnp.zeros((1,), jnp.int32), cmp.sum(axis=1).astype(jnp.int32)]
    )
    inv_order = (
        jnp.zeros((T,), jnp.int32)
        .at[order]
        .set(jnp.arange(T, dtype=jnp.int32))
    )

    w16 = jnp.broadcast_to(
        full_topk_weight.astype(jnp.float32).reshape(T, 1), (T, 128)
    )
    xs, ws16 = _make_sc_row_and_scalar_gather(T, K, 64)(x, order, w16)
    ys = _ragged_matmul(xs, W, ws16[:, :1], offsets_ext)
    out = _make_sc_row_gather(T, H, 32)(ys, inv_order)
    return out


# speedup vs baseline: 5.7187x; 1.0041x over previous
import functools

import jax
import jax.numpy as jnp
from jax import lax
from jax.experimental import pallas as pl
from jax.experimental.pallas import tpu as pltpu
from jax.experimental.pallas import tpu_sc as plsc

_SC_CORES = 2
_SC_SUBCORES = 16
_SC_WORKERS = _SC_CORES * _SC_SUBCORES

_BLK = 32


def _make_sc_row_gather(T_rows, D, R):
    per_w = T_rows // _SC_WORKERS
    n_chunks = per_w // R
    mesh = plsc.VectorSubcoreMesh(core_axis_name="c", subcore_axis_name="s")

    @functools.partial(
        pl.kernel,
        out_type=jax.ShapeDtypeStruct((T_rows, D), jnp.float32),
        mesh=mesh,
        scratch_types=[
            pltpu.VMEM((R,), jnp.int32),
            pltpu.VMEM((R, D), jnp.float32),
            pltpu.SemaphoreType.DMA,
        ],
    )
    def gather_kernel(src_hbm, idx_hbm, out_hbm, idx_v, rows_v, sem):
        wid = lax.axis_index("s") * _SC_CORES + lax.axis_index("c")
        for c in range(n_chunks):
            base = wid * per_w + c * R
            pltpu.sync_copy(idx_hbm.at[pl.ds(base, R)], idx_v)
            pltpu.async_copy(src_hbm.at[idx_v], rows_v, sem).wait()
            pltpu.sync_copy(rows_v, out_hbm.at[pl.ds(base, R)])

    return gather_kernel


def _make_sc_row_and_scalar_gather(T_rows, D, R):
    per_w = T_rows // _SC_WORKERS
    n_chunks = per_w // R
    lanes = 128
    mesh = plsc.VectorSubcoreMesh(core_axis_name="c", subcore_axis_name="s")

    @functools.partial(
        pl.kernel,
        out_type=(
            jax.ShapeDtypeStruct((T_rows, D), jnp.float32),
            jax.ShapeDtypeStruct((T_rows, lanes), jnp.float32),
        ),
        mesh=mesh,
        scratch_types=[
            pltpu.VMEM((R,), jnp.int32),
            pltpu.VMEM((R, D), jnp.float32),
            pltpu.VMEM((R, lanes), jnp.float32),
            pltpu.SemaphoreType.DMA,
            pltpu.SemaphoreType.DMA,
        ],
    )
    def gather_kernel(src_hbm, idx_hbm, svec_hbm, rows_out, s_out,
                      idx_v, rows_v, sg_v, sem, sem2):
        wid = lax.axis_index("s") * _SC_CORES + lax.axis_index("c")
        for c in range(n_chunks):
            base = wid * per_w + c * R
            pltpu.sync_copy(idx_hbm.at[pl.ds(base, R)], idx_v)
            cp = pltpu.async_copy(src_hbm.at[idx_v], rows_v, sem)
            cp2 = pltpu.async_copy(svec_hbm.at[idx_v], sg_v, sem2)
            cp.wait()
            cp2.wait()
            pltpu.sync_copy(rows_v, rows_out.at[pl.ds(base, R)])
            pltpu.sync_copy(sg_v, s_out.at[pl.ds(base, R)])

    return gather_kernel


def _ragged_matmul_body(off_r, ord_r, xs_ref, w_ref, ws_ref, o_ref):
    e = pl.program_id(0)
    start = off_r[e]
    end = off_r[e + 1]
    blk0 = start // _BLK
    n_chunks = (end + _BLK - 1) // _BLK - blk0

    def chunk(j, carry):
        s0 = (blk0 + j) * _BLK
        y = jnp.dot(
            xs_ref[pl.ds(s0, _BLK), :], w_ref[0],
            preferred_element_type=jnp.float32,
        )
        y = y * ws_ref[pl.ds(s0, _BLK), :]
        for i in range(_BLK):
            g = s0 + i

            @pl.when((g >= start) & (g < end))
            def _():
                o_ref[pl.ds(ord_r[g], 1), :] = y[i : i + 1, :]

        return carry

    lax.fori_loop(0, n_chunks, chunk, 0)


def _ragged_matmul(xs, W, ws, offsets_ext, order):
    T, K = xs.shape
    E, _, H = W.shape
    grid_spec = pltpu.PrefetchScalarGridSpec(
        num_scalar_prefetch=2,
        grid=(E,),
        in_specs=[
            pl.BlockSpec((T, K), lambda e, off, ordr: (0, 0)),
            pl.BlockSpec((1, K, H), lambda e, off, ordr: (e, 0, 0)),
            pl.BlockSpec((T, 1), lambda e, off, ordr: (0, 0)),
        ],
        out_specs=pl.BlockSpec((T, H), lambda e, off, ordr: (0, 0)),
    )
    return pl.pallas_call(
        _ragged_matmul_body,
        grid_spec=grid_spec,
        out_shape=jax.ShapeDtypeStruct((T, H), jnp.float32),
    )(offsets_ext, order, xs, W, ws)


def kernel(intermediate_states, down_weight, full_topk_ids, full_topk_weight):
    x = intermediate_states
    W = down_weight
    T, K = x.shape
    E, _, H = W.shape

    flat_ids = full_topk_ids.reshape(T).astype(jnp.int32)
    order = jnp.argsort(flat_ids).astype(jnp.int32)
    cmp = flat_ids[None, :] < jnp.arange(1, E + 1, dtype=jnp.int32)[:, None]
    offsets_ext = jnp.concatenate(
        [jnp.zeros((1,), jnp.int32), cmp.sum(axis=1).astype(jnp.int32)]
    )
    w16 = jnp.broadcast_to(
        full_topk_weight.astype(jnp.float32).reshape(T, 1), (T, 128)
    )
    xs, ws16 = _make_sc_row_and_scalar_gather(T, K, 64)(x, order, w16)
    out = _ragged_matmul(xs, W, ws16[:, :1], offsets_ext, order)
    return out
